# baseline (speedup 1.0000x reference)
import functools

import jax
import jax.numpy as jnp
from jax import lax
from jax.experimental import pallas as pl
from jax.experimental.pallas import tpu as pltpu
from jax.experimental.pallas import tpu_sc as plsc

N0 = 10000; N1 = 2500; E0 = 320000; E1 = 40000
D = 128; H = 128; C = 16
NC, NS, L = 2, 16, 16
NW = NC * NS
NP0 = 10240
NP1 = 2560
CH = 128
K0 = 80
K1 = 10
KP = 4
E0P = NW * K0 * CH
E1P = NW * K1 * CH
EPP = NW * KP * CH
GCH = 80
GK = NP0 // (NW * GCH)

_MESH = dict(core_axis_name="c", subcore_axis_name="s")
_SC_PARAMS = pltpu.CompilerParams(needs_layout_passes=False)
_BN_SCALE = 1.0 / (1.0 + 1e-5) ** 0.5


def _make_counts(e0t, e1t):
    mesh = plsc.VectorSubcoreMesh(**_MESH)

    @functools.partial(
        pl.kernel,
        out_type=(jax.ShapeDtypeStruct((NW, NP0), jnp.float32),
                  jax.ShapeDtypeStruct((NW, NP1), jnp.float32)),
        mesh=mesh,
        compiler_params=_SC_PARAMS,
        scratch_types=[
            pltpu.VMEM((e0t,), jnp.int32),
            pltpu.VMEM((e1t,), jnp.int32),
            pltpu.VMEM((NP0,), jnp.float32),
            pltpu.VMEM((NP1,), jnp.float32),
        ],
    )
    def cnt(dst0_hbm, dst1_hbm, c0_hbm, c1_hbm, i0, i1, t0, t1):
        cid = lax.axis_index("c")
        sid = lax.axis_index("s")
        wid = sid * NC + cid
        pltpu.sync_copy(dst0_hbm.at[wid], i0)
        pltpu.sync_copy(dst1_hbm.at[wid], i1)
        zv = jnp.zeros((L,), jnp.float32)

        def z0(i, carry):
            t0[pl.ds(i * L, L)] = zv
            return carry
        lax.fori_loop(0, NP0 // L, z0, 0)

        def z1(i, carry):
            t1[pl.ds(i * L, L)] = zv
            return carry
        lax.fori_loop(0, NP1 // L, z1, 0)

        ones = jnp.ones((L,), jnp.float32)

        def a0(e, carry):
            plsc.addupdate_scatter(t0, [i0[pl.ds(e * L, L)]], ones)
            return carry
        lax.fori_loop(0, e0t // L, a0, 0)

        def a1(e, carry):
            plsc.addupdate_scatter(t1, [i1[pl.ds(e * L, L)]], ones)
            return carry
        lax.fori_loop(0, e1t // L, a1, 0)

        pltpu.sync_copy(t0, c0_hbm.at[wid])
        pltpu.sync_copy(t1, c1_hbm.at[wid])

    return cnt


def _make_edge_scatter(n_tab, width, k, phases=1, tc_tiling=True):
    mesh = plsc.VectorSubcoreMesh(**_MESH)
    rpt = n_tab // NS
    kv = k // phases

    @functools.partial(
        pl.kernel,
        out_type=jax.ShapeDtypeStruct((NC, n_tab, width), jnp.float32),
        mesh=mesh,
        compiler_params=pltpu.CompilerParams(
            needs_layout_passes=False, use_tc_tiling_on_sc=tc_tiling),
        scratch_types=[
            pltpu.VMEM((kv, CH), jnp.int32),
            pltpu.VMEM((kv, CH), jnp.int32),
            pltpu.VMEM((CH, width), jnp.float32),
            pltpu.VMEM((CH, width), jnp.float32),
            pltpu.VMEM_SHARED((n_tab, width), jnp.float32),
            pltpu.SemaphoreType.DMA,
            pltpu.SemaphoreType.DMA,
        ],
    )
    def scat(y_hbm, srcs_hbm, dsts_hbm, zeros_hbm, out_hbm,
             src_v, dst_v, buf0, buf1, table, sem0, sem1):
        cid = lax.axis_index("c")
        sid = lax.axis_index("s")
        wid = sid * NC + cid
        off = sid * rpt
        for z in range(rpt // CH):
            pltpu.sync_copy(zeros_hbm, table.at[pl.ds(off + z * CH, CH)])
        if rpt % CH:
            pltpu.sync_copy(zeros_hbm.at[pl.ds(0, rpt % CH)],
                            table.at[pl.ds(off + (rpt // CH) * CH, rpt % CH)])
        plsc.subcore_barrier()

        bufs = (buf0, buf1)
        sems = (sem0, sem1)

        def wait(b):
            pltpu.make_async_copy(y_hbm.at[pl.ds(0, CH)], bufs[b], sems[b]).wait()

        for p in range(phases):
            pltpu.sync_copy(srcs_hbm.at[wid].at[pl.ds(p * kv, kv)], src_v)
            pltpu.sync_copy(dsts_hbm.at[wid].at[pl.ds(p * kv, kv)], dst_v)
            pltpu.async_copy(y_hbm.at[src_v.at[0]], buf0, sem0)

            def body(jj, carry):
                j = 2 * jj
                pltpu.async_copy(y_hbm.at[src_v.at[j + 1]], buf1, sem1)
                wait(0)
                pltpu.sync_copy(buf0, table.at[dst_v.at[j]], add=True)

                @pl.when(jj + 1 < kv // 2)
                def _issue():
                    pltpu.async_copy(y_hbm.at[src_v.at[j + 2]], buf0, sem0)

                wait(1)
                pltpu.sync_copy(buf1, table.at[dst_v.at[j + 1]], add=True)
                return carry
            lax.fori_loop(0, kv // 2, body, 0)

        plsc.subcore_barrier()
        pltpu.sync_copy(table.at[pl.ds(off, rpt)],
                        out_hbm.at[cid].at[pl.ds(off, rpt)])

    return scat


def _make_gather():
    mesh = plsc.VectorSubcoreMesh(**_MESH)

    @functools.partial(
        pl.kernel,
        out_type=jax.ShapeDtypeStruct((NP0, H), jnp.float32),
        mesh=mesh,
        compiler_params=_SC_PARAMS,
        scratch_types=[
            pltpu.VMEM((GK, GCH), jnp.int32),
            pltpu.VMEM((GCH, H), jnp.float32),
            pltpu.VMEM((GCH, H), jnp.float32),
            pltpu.SemaphoreType.DMA,
            pltpu.SemaphoreType.DMA,
        ],
    )
    def gath(x1_hbm, idx_hbm, out_hbm, idx_v, buf0, buf1, sem0, sem1):
        cid = lax.axis_index("c")
        sid = lax.axis_index("s")
        wid = sid * NC + cid
        base = wid * (GK * GCH)
        pltpu.sync_copy(idx_hbm.at[wid], idx_v)
        bufs = (buf0, buf1)
        sems = (sem0, sem1)
        pltpu.async_copy(x1_hbm.at[idx_v.at[0]], buf0, sem0)
        for j in range(GK):
            if j + 1 < GK:
                pltpu.async_copy(x1_hbm.at[idx_v.at[j + 1]],
                                 bufs[(j + 1) % 2], sems[(j + 1) % 2])
            pltpu.make_async_copy(x1_hbm.at[pl.ds(0, GCH)],
                                  bufs[j % 2], sems[j % 2]).wait()
            pltpu.sync_copy(bufs[j % 2], out_hbm.at[pl.ds(base + j * GCH, GCH)])

    return gath


_sc_counts = _make_counts(E0P // NW, E1P // NW)
_sc_scat0 = _make_edge_scatter(NP0, H, K0, phases=2)
_sc_scatp = _make_edge_scatter(NP1, H, KP)
_sc_scat1 = _make_edge_scatter(NP1, H, K1)
_sc_scat2 = _make_edge_scatter(NP0, C, K0, tc_tiling=False)
_sc_gather = _make_gather()


_BR = 1024
_G0 = NP0 // _BR


def _dis(c_ref):
    cntsum = jnp.sum(c_ref[...], axis=0)
    return lax.rsqrt(1.0 + cntsum)


def _tc_y0(x_p, W0, cnt0):
    def body(x_ref, w_ref, c_ref, y_ref):
        d = _dis(c_ref)
        y_ref[...] = jnp.dot(x_ref[...], w_ref[...],
                             preferred_element_type=jnp.float32) * d
    return pl.pallas_call(
        body,
        grid=(_G0,),
        in_specs=[pl.BlockSpec((_BR, D), lambda i: (i, 0)),
                  pl.BlockSpec((D, H), lambda i: (0, 0)),
                  pl.BlockSpec((NW, _BR, 1), lambda i: (0, i, 0))],
        out_specs=pl.BlockSpec((_BR, H), lambda i: (i, 0)),
        out_shape=jax.ShapeDtypeStruct((NP0, H), jnp.float32),
    )(x_p, W0, cnt0)


def _tc_combine(S, y, b, cnt, n, br):
    g = n // br

    def body(s_ref, y_ref, b_ref, c_ref, o_ref):
        d = _dis(c_ref)
        s = s_ref[0] + s_ref[1]
        o_ref[...] = jnp.maximum(d * (s + y_ref[...]) + b_ref[...], 0.0)
    return pl.pallas_call(
        body,
        grid=(g,),
        in_specs=[pl.BlockSpec((NC, br, H), lambda i: (0, i, 0)),
                  pl.BlockSpec((br, H), lambda i: (i, 0)),
                  pl.BlockSpec((1, H), lambda i: (0, 0)),
                  pl.BlockSpec((NW, br, 1), lambda i: (0, i, 0))],
        out_specs=pl.BlockSpec((br, H), lambda i: (i, 0)),
        out_shape=jax.ShapeDtypeStruct((n, H), jnp.float32),
    )(S, y, b, cnt)


def _tc_pool_mid(aggP, Pw0, Pb0, g0, bt0, W1, cnt1):
    def body(a_ref, pw_ref, pb_ref, g_ref, bt_ref, w_ref, c_ref, y_ref):
        d = _dis(c_ref)
        agg = a_ref[0] + a_ref[1]
        h = jnp.maximum(jnp.dot(agg, pw_ref[...],
                                preferred_element_type=jnp.float32)
                        + pb_ref[...], 0.0)
        xp = jnp.maximum(g_ref[...] * h * _BN_SCALE + bt_ref[...], 0.0)
        y_ref[...] = jnp.dot(xp, w_ref[...],
                             preferred_element_type=jnp.float32) * d
    return pl.pallas_call(
        body,
        grid=(1,),
        in_specs=[pl.BlockSpec((NC, NP1, H), lambda i: (0, 0, 0)),
                  pl.BlockSpec((H, H), lambda i: (0, 0)),
                  pl.BlockSpec((1, H), lambda i: (0, 0)),
                  pl.BlockSpec((1, H), lambda i: (0, 0)),
                  pl.BlockSpec((1, H), lambda i: (0, 0)),
                  pl.BlockSpec((H, H), lambda i: (0, 0)),
                  pl.BlockSpec((NW, NP1, 1), lambda i: (0, 0, 0))],
        out_specs=pl.BlockSpec((NP1, H), lambda i: (0, 0)),
        out_shape=jax.ShapeDtypeStruct((NP1, H), jnp.float32),
    )(aggP, Pw0, Pb0, g0, bt0, W1, cnt1)


def _tc_up(g, x0, Pw1, Pb1, g1, bt1, W2a, W2b, cnt0):
    def body(g_ref, x0_ref, pw_ref, pb_ref, gm_ref, bt_ref,
             wa_ref, wb_ref, c_ref, y_ref):
        d = _dis(c_ref)
        h = jnp.maximum(jnp.dot(g_ref[...], pw_ref[...],
                                preferred_element_type=jnp.float32)
                        + pb_ref[...], 0.0)
        xu = jnp.maximum(gm_ref[...] * h * _BN_SCALE + bt_ref[...], 0.0)
        z = (jnp.dot(xu, wa_ref[...], preferred_element_type=jnp.float32)
             + jnp.dot(x0_ref[...], wb_ref[...],
                       preferred_element_type=jnp.float32))
        y_ref[...] = z * d
    return pl.pallas_call(
        body,
        grid=(_G0,),
        in_specs=[pl.BlockSpec((_BR, H), lambda i: (i, 0)),
                  pl.BlockSpec((_BR, H), lambda i: (i, 0)),
                  pl.BlockSpec((H, H), lambda i: (0, 0)),
                  pl.BlockSpec((1, H), lambda i: (0, 0)),
                  pl.BlockSpec((1, H), lambda i: (0, 0)),
                  pl.BlockSpec((1, H), lambda i: (0, 0)),
                  pl.BlockSpec((H, C), lambda i: (0, 0)),
                  pl.BlockSpec((H, C), lambda i: (0, 0)),
                  pl.BlockSpec((NW, _BR, 1), lambda i: (0, i, 0))],
        out_specs=pl.BlockSpec((_BR, C), lambda i: (i, 0)),
        out_shape=jax.ShapeDtypeStruct((NP0, C), jnp.float32),
    )(g, x0, Pw1, Pb1, g1, bt1, W2a, W2b, cnt0)


def _tc_final(S2, y2, b2, cnt0):
    def body(s_ref, y_ref, b_ref, c_ref, o_ref):
        d = _dis(c_ref)
        z = d * (s_ref[0] + s_ref[1] + y_ref[...]) + b_ref[...]
        m = jnp.max(z, axis=1, keepdims=True)
        lse = m + jnp.log(jnp.sum(jnp.exp(z - m), axis=1, keepdims=True))
        o_ref[...] = z - lse
    return pl.pallas_call(
        body,
        grid=(_G0,),
        in_specs=[pl.BlockSpec((NC, _BR, C), lambda i: (0, i, 0)),
                  pl.BlockSpec((_BR, C), lambda i: (i, 0)),
                  pl.BlockSpec((1, C), lambda i: (0, 0)),
                  pl.BlockSpec((NW, _BR, 1), lambda i: (0, i, 0))],
        out_specs=pl.BlockSpec((_BR, C), lambda i: (i, 0)),
        out_shape=jax.ShapeDtypeStruct((NP0, C), jnp.float32),
    )(S2, y2, b2, cnt0)


def kernel(x, W0, b0, W1, b1, W2, b2, Pw0, Pb0, g0, bt0,
           Pw1, Pb1, g1, bt1, edge_index, edge_index_l1, inter_edge):
    i32 = jnp.int32
    ei = edge_index.astype(i32)
    ei1 = edge_index_l1.astype(i32)
    it = inter_edge.astype(i32)

    x_p = jnp.zeros((NP0, D), jnp.float32).at[:N0].set(x)

    def _pad_dst(n_pad, lo, hi):
        return lo + jnp.arange(n_pad, dtype=i32) % (hi - lo)

    pad0s = jnp.full((E0P - E0,), N0, i32)
    pad0d = _pad_dst(E0P - E0, N0, NP0)
    src0 = jnp.concatenate([ei[0], pad0s]).reshape(NW, K0, CH)
    dst0 = jnp.concatenate([ei[1], pad0d]).reshape(NW, K0, CH)
    src1 = jnp.concatenate([ei1[0], jnp.full((E1P - E1,), N1, i32)]
                           ).reshape(NW, K1, CH)
    dst1 = jnp.concatenate([ei1[1], _pad_dst(E1P - E1, N1, NP1)]
                           ).reshape(NW, K1, CH)
    srcp = jnp.concatenate([it[0], jnp.full((EPP - N0,), N0, i32)]
                           ).reshape(NW, KP, CH)
    dstp = jnp.concatenate([it[1], _pad_dst(EPP - N0, N1, NP1)]
                           ).reshape(NW, KP, CH)
    idxu = jnp.concatenate([it[1], jnp.zeros((NP0 - N0,), i32)]
                           ).reshape(NW, GK, GCH)

    z128 = jnp.zeros((CH, H), jnp.float32)
    z16 = jnp.zeros((CH, C), jnp.float32)
    b0r = b0.reshape(1, H); b1r = b1.reshape(1, H); b2r = b2.reshape(1, C)
    pb0r = Pb0.reshape(1, H); g0r = g0.reshape(1, H); bt0r = bt0.reshape(1, H)
    pb1r = Pb1.reshape(1, H); g1r = g1.reshape(1, H); bt1r = bt1.reshape(1, H)
    W2a = W2[:H]; W2b = W2[H:]

    c0p, c1p = _sc_counts(dst0.reshape(NW, -1), dst1.reshape(NW, -1))
    cnt0 = c0p[..., None]
    cnt1 = c1p[..., None]

    y0 = _tc_y0(x_p, W0, cnt0)
    S0 = _sc_scat0(y0, src0, dst0, z128)
    x0 = _tc_combine(S0, y0, b0r, cnt0, NP0, _BR)
    aggP = _sc_scatp(x0, srcp, dstp, z128)
    y1 = _tc_pool_mid(aggP, Pw0, pb0r, g0r, bt0r, W1, cnt1)
    S1 = _sc_scat1(y1, src1, dst1, z128)
    x1 = _tc_combine(S1, y1, b1r, cnt1, NP1, NP1)
    g = _sc_gather(x1, idxu)
    y2 = _tc_up(g, x0, Pw1, pb1r, g1r, bt1r, W2a, W2b, cnt0)
    S2 = _sc_scat2(y2, src0, dst0, z16)
    outp = _tc_final(S2, y2, b2r, cnt0)
    return outp[:N0]


# speedup vs baseline: 1.2415x; 1.2415x over previous
import functools

import jax
import jax.numpy as jnp
from jax import lax
from jax.experimental import pallas as pl
from jax.experimental.pallas import tpu as pltpu
from jax.experimental.pallas import tpu_sc as plsc

N0 = 10000; N1 = 2500; E0 = 320000; E1 = 40000
D = 128; H = 128; C = 16
NC, NS, L = 2, 16, 16
NW = NC * NS
NP0 = 10240
NP1 = 2560
CH = 128
K0 = 80
K1 = 10
KP = 4
E0P = NW * K0 * CH
E1P = NW * K1 * CH
EPP = NW * KP * CH
GCH = 80
GK = NP0 // (NW * GCH)

_MESH = dict(core_axis_name="c", subcore_axis_name="s")
_SC_PARAMS = pltpu.CompilerParams(needs_layout_passes=False)
_BN_SCALE = 1.0 / (1.0 + 1e-5) ** 0.5


def _make_counts(e0t, e1t):
    mesh = plsc.VectorSubcoreMesh(**_MESH)

    @functools.partial(
        pl.kernel,
        out_type=(jax.ShapeDtypeStruct((NW, NP0), jnp.float32),
                  jax.ShapeDtypeStruct((NW, NP1), jnp.float32)),
        mesh=mesh,
        compiler_params=_SC_PARAMS,
        scratch_types=[
            pltpu.VMEM((e0t,), jnp.int32),
            pltpu.VMEM((e1t,), jnp.int32),
            pltpu.VMEM((NP0,), jnp.float32),
            pltpu.VMEM((NP1,), jnp.float32),
        ],
    )
    def cnt(dst0_hbm, dst1_hbm, c0_hbm, c1_hbm, i0, i1, t0, t1):
        cid = lax.axis_index("c")
        sid = lax.axis_index("s")
        wid = sid * NC + cid
        pltpu.sync_copy(dst0_hbm.at[wid], i0)
        pltpu.sync_copy(dst1_hbm.at[wid], i1)
        zv = jnp.zeros((L,), jnp.float32)

        def z0(i, carry):
            t0[pl.ds(i * L, L)] = zv
            return carry
        lax.fori_loop(0, NP0 // L, z0, 0)

        def z1(i, carry):
            t1[pl.ds(i * L, L)] = zv
            return carry
        lax.fori_loop(0, NP1 // L, z1, 0)

        ones = jnp.ones((L,), jnp.float32)

        def a0(e, carry):
            plsc.addupdate_scatter(t0, [i0[pl.ds(e * L, L)]], ones)
            return carry
        lax.fori_loop(0, e0t // L, a0, 0)

        def a1(e, carry):
            plsc.addupdate_scatter(t1, [i1[pl.ds(e * L, L)]], ones)
            return carry
        lax.fori_loop(0, e1t // L, a1, 0)

        pltpu.sync_copy(t0, c0_hbm.at[wid])
        pltpu.sync_copy(t1, c1_hbm.at[wid])

    return cnt


def _make_edge_scatter(n_tab, width, k, phases=1, tc_tiling=True,
                       linear_src=False, n_src_max=0):
    mesh = plsc.VectorSubcoreMesh(**_MESH)
    rpt = n_tab // NS
    kv = k // phases

    @functools.partial(
        pl.kernel,
        out_type=jax.ShapeDtypeStruct((NC, n_tab, width), jnp.float32),
        mesh=mesh,
        compiler_params=pltpu.CompilerParams(
            needs_layout_passes=False, use_tc_tiling_on_sc=tc_tiling),
        scratch_types=[
            pltpu.VMEM((kv, CH), jnp.int32),
            pltpu.VMEM((kv, CH), jnp.int32),
            pltpu.VMEM((CH, width), jnp.float32),
            pltpu.VMEM((CH, width), jnp.float32),
            pltpu.VMEM_SHARED((n_tab, width), jnp.float32),
            pltpu.SemaphoreType.DMA,
            pltpu.SemaphoreType.DMA,
        ],
    )
    def scat(y_hbm, srcs_hbm, dsts_hbm, zeros_hbm, out_hbm,
             src_v, dst_v, buf0, buf1, table, sem0, sem1):
        cid = lax.axis_index("c")
        sid = lax.axis_index("s")
        wid = sid * NC + cid
        off = sid * rpt
        for z in range(rpt // CH):
            pltpu.sync_copy(zeros_hbm, table.at[pl.ds(off + z * CH, CH)])
        if rpt % CH:
            pltpu.sync_copy(zeros_hbm.at[pl.ds(0, rpt % CH)],
                            table.at[pl.ds(off + (rpt // CH) * CH, rpt % CH)])
        plsc.subcore_barrier()

        bufs = (buf0, buf1)
        sems = (sem0, sem1)

        def wait(b):
            pltpu.make_async_copy(y_hbm.at[pl.ds(0, CH)], bufs[b], sems[b]).wait()

        ebase = wid * k * CH

        def gath(p, j, b):
            if linear_src:
                row = jnp.minimum(ebase + (p * kv + j) * CH, n_src_max - CH)
                pltpu.async_copy(y_hbm.at[pl.ds(row, CH)], bufs[b], sems[b])
            else:
                pltpu.async_copy(y_hbm.at[src_v.at[j]], bufs[b], sems[b])

        for p in range(phases):
            if not linear_src:
                pltpu.sync_copy(srcs_hbm.at[wid].at[pl.ds(p * kv, kv)], src_v)
            pltpu.sync_copy(dsts_hbm.at[wid].at[pl.ds(p * kv, kv)], dst_v)
            gath(p, 0, 0)

            def body(jj, carry):
                j = 2 * jj
                gath(p, j + 1, 1)
                wait(0)
                pltpu.sync_copy(buf0, table.at[dst_v.at[j]], add=True)

                @pl.when(jj + 1 < kv // 2)
                def _issue():
                    gath(p, j + 2, 0)

                wait(1)
                pltpu.sync_copy(buf1, table.at[dst_v.at[j + 1]], add=True)
                return carry
            lax.fori_loop(0, kv // 2, body, 0)

        plsc.subcore_barrier()
        pltpu.sync_copy(table.at[pl.ds(off, rpt)],
                        out_hbm.at[cid].at[pl.ds(off, rpt)])

    return scat


def _make_gather():
    mesh = plsc.VectorSubcoreMesh(**_MESH)

    @functools.partial(
        pl.kernel,
        out_type=jax.ShapeDtypeStruct((NP0, H), jnp.float32),
        mesh=mesh,
        compiler_params=_SC_PARAMS,
        scratch_types=[
            pltpu.VMEM((GK, GCH), jnp.int32),
            pltpu.VMEM((GCH, H), jnp.float32),
            pltpu.VMEM((GCH, H), jnp.float32),
            pltpu.SemaphoreType.DMA,
            pltpu.SemaphoreType.DMA,
        ],
    )
    def gath(x1_hbm, idx_hbm, out_hbm, idx_v, buf0, buf1, sem0, sem1):
        cid = lax.axis_index("c")
        sid = lax.axis_index("s")
        wid = sid * NC + cid
        base = wid * (GK * GCH)
        pltpu.sync_copy(idx_hbm.at[wid], idx_v)
        bufs = (buf0, buf1)
        sems = (sem0, sem1)
        pltpu.async_copy(x1_hbm.at[idx_v.at[0]], buf0, sem0)
        for j in range(GK):
            if j + 1 < GK:
                pltpu.async_copy(x1_hbm.at[idx_v.at[j + 1]],
                                 bufs[(j + 1) % 2], sems[(j + 1) % 2])
            pltpu.make_async_copy(x1_hbm.at[pl.ds(0, GCH)],
                                  bufs[j % 2], sems[j % 2]).wait()
            pltpu.sync_copy(bufs[j % 2], out_hbm.at[pl.ds(base + j * GCH, GCH)])

    return gath


_sc_counts = _make_counts(E0P // NW, E1P // NW)
_sc_scat0 = _make_edge_scatter(NP0, H, K0, phases=2)
_sc_scatp = _make_edge_scatter(NP1, H, KP, linear_src=True, n_src_max=NP0)
_sc_scat1 = _make_edge_scatter(NP1, H, K1)
_sc_scat2 = _make_edge_scatter(NP0, C, K0, tc_tiling=False)
_sc_gather = _make_gather()


_BR = 1024
_G0 = NP0 // _BR


def _dis(c_ref):
    cntsum = jnp.sum(c_ref[...], axis=0)
    return lax.rsqrt(1.0 + cntsum)


def _tc_y0(x_p, W0, cnt0):
    def body(x_ref, w_ref, c_ref, y_ref):
        d = _dis(c_ref)
        y_ref[...] = jnp.dot(x_ref[...], w_ref[...],
                             preferred_element_type=jnp.float32) * d
    return pl.pallas_call(
        body,
        grid=(_G0,),
        in_specs=[pl.BlockSpec((_BR, D), lambda i: (i, 0)),
                  pl.BlockSpec((D, H), lambda i: (0, 0)),
                  pl.BlockSpec((NW, _BR, 1), lambda i: (0, i, 0))],
        out_specs=pl.BlockSpec((_BR, H), lambda i: (i, 0)),
        out_shape=jax.ShapeDtypeStruct((NP0, H), jnp.float32),
    )(x_p, W0, cnt0)


def _tc_combine(S, y, b, cnt, n, br):
    g = n // br

    def body(s_ref, y_ref, b_ref, c_ref, o_ref):
        d = _dis(c_ref)
        s = s_ref[0] + s_ref[1]
        o_ref[...] = jnp.maximum(d * (s + y_ref[...]) + b_ref[...], 0.0)
    return pl.pallas_call(
        body,
        grid=(g,),
        in_specs=[pl.BlockSpec((NC, br, H), lambda i: (0, i, 0)),
                  pl.BlockSpec((br, H), lambda i: (i, 0)),
                  pl.BlockSpec((1, H), lambda i: (0, 0)),
                  pl.BlockSpec((NW, br, 1), lambda i: (0, i, 0))],
        out_specs=pl.BlockSpec((br, H), lambda i: (i, 0)),
        out_shape=jax.ShapeDtypeStruct((n, H), jnp.float32),
    )(S, y, b, cnt)


def _tc_pool_mid(aggP, Pw0, Pb0, g0, bt0, W1, cnt1):
    def body(a_ref, pw_ref, pb_ref, g_ref, bt_ref, w_ref, c_ref, y_ref):
        d = _dis(c_ref)
        agg = a_ref[0] + a_ref[1]
        h = jnp.maximum(jnp.dot(agg, pw_ref[...],
                                preferred_element_type=jnp.float32)
                        + pb_ref[...], 0.0)
        xp = jnp.maximum(g_ref[...] * h * _BN_SCALE + bt_ref[...], 0.0)
        y_ref[...] = jnp.dot(xp, w_ref[...],
                             preferred_element_type=jnp.float32) * d
    return pl.pallas_call(
        body,
        grid=(1,),
        in_specs=[pl.BlockSpec((NC, NP1, H), lambda i: (0, 0, 0)),
                  pl.BlockSpec((H, H), lambda i: (0, 0)),
                  pl.BlockSpec((1, H), lambda i: (0, 0)),
                  pl.BlockSpec((1, H), lambda i: (0, 0)),
                  pl.BlockSpec((1, H), lambda i: (0, 0)),
                  pl.BlockSpec((H, H), lambda i: (0, 0)),
                  pl.BlockSpec((NW, NP1, 1), lambda i: (0, 0, 0))],
        out_specs=pl.BlockSpec((NP1, H), lambda i: (0, 0)),
        out_shape=jax.ShapeDtypeStruct((NP1, H), jnp.float32),
    )(aggP, Pw0, Pb0, g0, bt0, W1, cnt1)


def _tc_up(g, x0, Pw1, Pb1, g1, bt1, W2a, W2b, cnt0):
    def body(g_ref, x0_ref, pw_ref, pb_ref, gm_ref, bt_ref,
             wa_ref, wb_ref, c_ref, y_ref):
        d = _dis(c_ref)
        h = jnp.maximum(jnp.dot(g_ref[...], pw_ref[...],
                                preferred_element_type=jnp.float32)
                        + pb_ref[...], 0.0)
        xu = jnp.maximum(gm_ref[...] * h * _BN_SCALE + bt_ref[...], 0.0)
        z = (jnp.dot(xu, wa_ref[...], preferred_element_type=jnp.float32)
             + jnp.dot(x0_ref[...], wb_ref[...],
                       preferred_element_type=jnp.float32))
        y_ref[...] = z * d
    return pl.pallas_call(
        body,
        grid=(_G0,),
        in_specs=[pl.BlockSpec((_BR, H), lambda i: (i, 0)),
                  pl.BlockSpec((_BR, H), lambda i: (i, 0)),
                  pl.BlockSpec((H, H), lambda i: (0, 0)),
                  pl.BlockSpec((1, H), lambda i: (0, 0)),
                  pl.BlockSpec((1, H), lambda i: (0, 0)),
                  pl.BlockSpec((1, H), lambda i: (0, 0)),
                  pl.BlockSpec((H, C), lambda i: (0, 0)),
                  pl.BlockSpec((H, C), lambda i: (0, 0)),
                  pl.BlockSpec((NW, _BR, 1), lambda i: (0, i, 0))],
        out_specs=pl.BlockSpec((_BR, C), lambda i: (i, 0)),
        out_shape=jax.ShapeDtypeStruct((NP0, C), jnp.float32),
    )(g, x0, Pw1, Pb1, g1, bt1, W2a, W2b, cnt0)


def _tc_final(S2, y2, b2, cnt0):
    def body(s_ref, y_ref, b_ref, c_ref, o_ref):
        d = _dis(c_ref)
        z = d * (s_ref[0] + s_ref[1] + y_ref[...]) + b_ref[...]
        m = jnp.max(z, axis=1, keepdims=True)
        lse = m + jnp.log(jnp.sum(jnp.exp(z - m), axis=1, keepdims=True))
        o_ref[...] = z - lse
    return pl.pallas_call(
        body,
        grid=(_G0,),
        in_specs=[pl.BlockSpec((NC, _BR, C), lambda i: (0, i, 0)),
                  pl.BlockSpec((_BR, C), lambda i: (i, 0)),
                  pl.BlockSpec((1, C), lambda i: (0, 0)),
                  pl.BlockSpec((NW, _BR, 1), lambda i: (0, i, 0))],
        out_specs=pl.BlockSpec((_BR, C), lambda i: (i, 0)),
        out_shape=jax.ShapeDtypeStruct((NP0, C), jnp.float32),
    )(S2, y2, b2, cnt0)


def kernel(x, W0, b0, W1, b1, W2, b2, Pw0, Pb0, g0, bt0,
           Pw1, Pb1, g1, bt1, edge_index, edge_index_l1, inter_edge):
    i32 = jnp.int32
    ei = edge_index.astype(i32)
    ei1 = edge_index_l1.astype(i32)
    it = inter_edge.astype(i32)

    x_p = jnp.zeros((NP0, D), jnp.float32).at[:N0].set(x)

    def _pad_dst(n_pad, lo, hi):
        return lo + jnp.arange(n_pad, dtype=i32) % (hi - lo)

    pad0s = jnp.full((E0P - E0,), N0, i32)
    pad0d = _pad_dst(E0P - E0, N0, NP0)
    src0 = jnp.concatenate([ei[0], pad0s]).reshape(NW, K0, CH)
    dst0 = jnp.concatenate([ei[1], pad0d]).reshape(NW, K0, CH)
    src1 = jnp.concatenate([ei1[0], jnp.full((E1P - E1,), N1, i32)]
                           ).reshape(NW, K1, CH)
    dst1 = jnp.concatenate([ei1[1], _pad_dst(E1P - E1, N1, NP1)]
                           ).reshape(NW, K1, CH)
    srcp = jnp.concatenate([it[0], jnp.full((EPP - N0,), N0, i32)]
                           ).reshape(NW, KP, CH)
    dstp = jnp.concatenate([it[1], _pad_dst(EPP - N0, N1, NP1)]
                           ).reshape(NW, KP, CH)
    idxu = jnp.concatenate([it[1], jnp.zeros((NP0 - N0,), i32)]
                           ).reshape(NW, GK, GCH)

    z128 = jnp.zeros((CH, H), jnp.float32)
    z16 = jnp.zeros((CH, C), jnp.float32)
    b0r = b0.reshape(1, H); b1r = b1.reshape(1, H); b2r = b2.reshape(1, C)
    pb0r = Pb0.reshape(1, H); g0r = g0.reshape(1, H); bt0r = bt0.reshape(1, H)
    pb1r = Pb1.reshape(1, H); g1r = g1.reshape(1, H); bt1r = bt1.reshape(1, H)
    W2a = W2[:H]; W2b = W2[H:]

    c0p, c1p = _sc_counts(dst0.reshape(NW, -1), dst1.reshape(NW, -1))
    cnt0 = c0p[..., None]
    cnt1 = c1p[..., None]

    y0 = _tc_y0(x_p, W0, cnt0)
    S0 = _sc_scat0(y0, src0, dst0, z128)
    x0 = _tc_combine(S0, y0, b0r, cnt0, NP0, _BR)
    aggP = _sc_scatp(x0, srcp, dstp, z128)
    y1 = _tc_pool_mid(aggP, Pw0, pb0r, g0r, bt0r, W1, cnt1)
    S1 = _sc_scat1(y1, src1, dst1, z128)
    x1 = _tc_combine(S1, y1, b1r, cnt1, NP1, NP1)
    g = _sc_gather(x1, idxu)
    y2 = _tc_up(g, x0, Pw1, pb1r, g1r, bt1r, W2a, W2b, cnt0)
    S2 = _sc_scat2(y2, src0, dst0, z16)
    outp = _tc_final(S2, y2, b2r, cnt0)
    return outp[:N0]
